# baseline (speedup 1.0000x reference)
import jax
import jax.numpy as jnp
from jax.experimental import pallas as pl


def _s1_kernel(x_ref, w1_ref, o_ref):
    o_ref[...] = jnp.dot(x_ref[...], w1_ref[...],
                         preferred_element_type=jnp.float32)


def _pass1_kernel(adj_ref, s1_ref, b1_ref, w2_ref, s2_ref):
    h = jnp.dot(adj_ref[...], s1_ref[...],
                preferred_element_type=jnp.float32)
    h = jnp.maximum(h + b1_ref[...], 0.0)
    s2_ref[...] = jnp.dot(h, w2_ref[...],
                          preferred_element_type=jnp.float32)


def _pass2_kernel(adj_ref, s2_ref, b2_ref, o_ref):
    y = jnp.dot(adj_ref[...], s2_ref[...],
                preferred_element_type=jnp.float32) + b2_ref[...]
    m = jnp.max(y, axis=1, keepdims=True)
    z = y - m
    lse = jnp.log(jnp.sum(jnp.exp(z), axis=1, keepdims=True))
    o_ref[...] = z - lse


def kernel(x, adj, W1, b1, W2, b2):
    n, f_in = x.shape
    h = W1.shape[1]
    c = W2.shape[1]
    bm = 400

    s1 = pl.pallas_call(
        _s1_kernel,
        out_shape=jax.ShapeDtypeStruct((n, h), jnp.float32),
    )(x, W1)

    s2 = pl.pallas_call(
        _pass1_kernel,
        grid=(n // bm,),
        in_specs=[
            pl.BlockSpec((bm, n), lambda i: (i, 0)),
            pl.BlockSpec((n, h), lambda i: (0, 0)),
            pl.BlockSpec((1, h), lambda i: (0, 0)),
            pl.BlockSpec((h, c), lambda i: (0, 0)),
        ],
        out_specs=pl.BlockSpec((bm, c), lambda i: (i, 0)),
        out_shape=jax.ShapeDtypeStruct((n, c), jnp.float32),
    )(adj, s1, b1.reshape(1, h), W2)

    out = pl.pallas_call(
        _pass2_kernel,
        grid=(n // bm,),
        in_specs=[
            pl.BlockSpec((bm, n), lambda i: (i, 0)),
            pl.BlockSpec((n, c), lambda i: (0, 0)),
            pl.BlockSpec((1, c), lambda i: (0, 0)),
        ],
        out_specs=pl.BlockSpec((bm, c), lambda i: (i, 0)),
        out_shape=jax.ShapeDtypeStruct((n, c), jnp.float32),
    )(adj, s2, b2.reshape(1, c))

    return out
